# Initial kernel scaffold; baseline (speedup 1.0000x reference)
import jax
import jax.numpy as jnp
from jax.experimental import pallas as pl


def kernel(x, W_enc, b_enc):
    raise NotImplementedError("write your pallas kernel here")


# speedup vs baseline: 2.1004x; 2.1004x over previous
import jax
import jax.numpy as jnp
from jax import lax
from jax.experimental import pallas as pl
from jax.experimental.pallas import tpu as pltpu

_B, _T, _DIN, _DSAE, _K = 16, 2, 768, 65536, 128
_BLK = 2048
_NBLK = _DSAE // _BLK
_CH = 2048
_NCH = _DSAE // _CH
_MININT = -2147483648


def _body(x_ref, w_ref, b_ref, out_ref, key_scr, rm_scr):
    i = pl.program_id(0)
    pre0 = jnp.dot(x_ref[0], w_ref[0], preferred_element_type=jnp.float32)
    pre1 = jnp.dot(x_ref[1], w_ref[1], preferred_element_type=jnp.float32)
    psum = pre0 + pre1 + b_ref[...]
    pb = lax.bitcast_convert_type(psum, jnp.int32)
    key = jnp.where(pb < 0,
                    jnp.bitwise_xor(jnp.bitwise_not(pb), jnp.int32(_MININT)),
                    pb)
    key_scr[:, pl.ds(i * _BLK, _BLK)] = key
    rm_scr[:, pl.ds(i * _BLK, _BLK)] = 0.5 * (
        jnp.maximum(pre0, 0.0) + jnp.maximum(pre1, 0.0))

    @pl.when(i == _NBLK - 1)
    def _finalize():
        kk = jnp.int32(_K)

        def count_ge(t):
            def cbody(c, acc):
                kch = key_scr[:, pl.ds(c * _CH, _CH)]
                return acc + jnp.sum((kch >= t).astype(jnp.int32), axis=1,
                                     keepdims=True)
            return lax.fori_loop(0, _NCH, cbody,
                                 jnp.zeros((_B, 1), jnp.int32))

        thr = jnp.where(count_ge(jnp.zeros((_B, 1), jnp.int32)) >= kk,
                        jnp.int32(0), jnp.int32(_MININT))

        def bbody(it, thr):
            j = 30 - it
            cand = jnp.bitwise_or(thr, lax.shift_left(jnp.int32(1), j))
            return jnp.where(count_ge(cand) >= kk, cand, thr)

        thr = lax.fori_loop(0, 31, bbody, thr)

        def wbody(c, carry):
            kch = key_scr[:, pl.ds(c * _CH, _CH)]
            rch = rm_scr[:, pl.ds(c * _CH, _CH)]
            out_ref[:, pl.ds(c * _CH, _CH)] = jnp.where(kch >= thr, rch, 0.0)
            return carry

        lax.fori_loop(0, _NCH, wbody, 0)


def kernel(x, W_enc, b_enc):
    xt = jnp.transpose(x, (1, 0, 2))
    b2 = b_enc.reshape(1, _DSAE)
    return pl.pallas_call(
        _body,
        grid=(_NBLK,),
        in_specs=[
            pl.BlockSpec((_T, _B, _DIN), lambda i: (0, 0, 0)),
            pl.BlockSpec((_T, _DIN, _BLK), lambda i: (0, 0, i)),
            pl.BlockSpec((1, _BLK), lambda i: (0, i)),
        ],
        out_specs=pl.BlockSpec((_B, _DSAE), lambda i: (0, 0)),
        out_shape=jax.ShapeDtypeStruct((_B, _DSAE), jnp.float32),
        scratch_shapes=[
            pltpu.VMEM((_B, _DSAE), jnp.int32),
            pltpu.VMEM((_B, _DSAE), jnp.float32),
        ],
        compiler_params=pltpu.CompilerParams(
            dimension_semantics=("arbitrary",),
        ),
    )(xt, W_enc, b2)
